# baseline (speedup 1.0000x reference)
import functools

import jax
import jax.numpy as jnp
from jax.experimental import pallas as pl
from jax.experimental.pallas import tpu as pltpu

F32 = jnp.float32
_BN_EPS = 1e-5
_KNN = 8


def _knn_body(q_ref, pT_ref, ps_ref, f2_ref, w1a_ref, w1b_ref, b1_ref,
              x1_ref, warr_ref, cof_ref, clf_ref, st_ref):
    TN = q_ref.shape[1]
    M = pT_ref.shape[2]
    OC = f2_ref.shape[2] // 2
    q = q_ref[0]
    pT = pT_ref[0]
    ps = ps_ref[0]
    F2 = f2_ref[0]
    pn = jnp.sum(pT * pT, axis=0, keepdims=True)
    iota = jax.lax.broadcasted_iota(jnp.int32, (TN, M), 1)

    def amin(d2):
        m = jnp.min(d2, axis=1, keepdims=True)
        return jnp.min(jnp.where(d2 == m, iota, M), axis=1, keepdims=True)

    def dist2(v):
        acc = v[:, 0:1] * pT[0:1, :]
        acc = acc + v[:, 1:2] * pT[1:2, :]
        acc = acc + v[:, 2:3] * pT[2:3, :]
        return pn - 2.0 * acc

    def gather24(oh_bf):
        g = jnp.dot(oh_bf, ps, preferred_element_type=F32)
        return (g[:, 16:24] + g[:, 8:16]) + g[:, 0:8]

    ci = amin(dist2(q))
    ohcb = (iota == ci).astype(F32).astype(jnp.bfloat16)
    cp = gather24(ohcb)
    clf2 = jnp.dot(ohcb, F2, preferred_element_type=F32)
    clf_ref[0] = clf2[:, :OC] + clf2[:, OC:]

    d2 = dist2(cp)
    emat = jnp.zeros((TN, M), F32)
    es = []
    deltas = []
    for k in range(_KNN):
        ki = amin(d2)
        oh = iota == ki
        d2 = jnp.where(oh, jnp.inf, d2)
        if k == 0:
            continue
        ohf = oh.astype(F32)
        delta = gather24(ohf.astype(jnp.bfloat16)) - q
        dst = jnp.sqrt(jnp.sum(delta * delta, axis=1, keepdims=True))
        e = jnp.exp(-10.0 * dst)
        emat = emat + e * ohf
        es.append(e)
        deltas.append(delta)

    esum = es[0]
    for e in es[1:]:
        esum = esum + e
    inv = 1.0 / (esum + 1e-7)
    wmat = emat * inv
    wh = wmat.astype(jnp.bfloat16)
    wl = (wmat - wh.astype(F32)).astype(jnp.bfloat16)
    y1 = jnp.dot(wh, F2, preferred_element_type=F32)
    y2 = jnp.dot(wl, F2[:, :OC], preferred_element_type=F32)
    cof_ref[0] = (y1[:, :OC] + y1[:, OC:]) + y2
    warr_ref[0] = jnp.concatenate([esum * inv] + [e * inv for e in es], axis=1)

    base = jnp.dot(cp, w1b_ref[...], preferred_element_type=F32, precision=jax.lax.Precision.HIGHEST) + b1_ref[...]
    ssum = jnp.zeros((1, x1_ref.shape[-1]), F32)
    qsum = jnp.zeros((1, x1_ref.shape[-1]), F32)
    for k in range(_KNN - 1):
        x1 = jnp.dot(deltas[k], w1a_ref[...], preferred_element_type=F32, precision=jax.lax.Precision.HIGHEST) + base
        x1_ref[0, k] = x1
        ssum = ssum + jnp.sum(x1, axis=0, keepdims=True)
        qsum = qsum + jnp.sum(x1 * x1, axis=0, keepdims=True)

    first = jnp.logical_and(pl.program_id(0) == 0, pl.program_id(1) == 0)

    @pl.when(first)
    def _():
        st_ref[...] = jnp.zeros_like(st_ref)

    st_ref[...] = st_ref[...] + jnp.concatenate([ssum, qsum], axis=0)


def _bn_affine_in(st, rows, g, beta):
    mu = st[0:1] / rows
    var = jnp.maximum(st[1:2] / rows - mu * mu, 0.0)
    sc = g * jax.lax.rsqrt(var + _BN_EPS)
    sh = beta - mu * sc
    return sc, sh


def _mlp_body(x_ref, st_ref_in, g_ref, bt_ref, w_ref, b_ref, *out_refs,
              act, stats, rows):
    if act:
        sc, sh = _bn_affine_in(st_ref_in[...], rows, g_ref[...], bt_ref[...])
        h = jnp.maximum(x_ref[...] * sc + sh, 0.0)
    else:
        h = x_ref[...]
    y = jnp.dot(h, w_ref[...], preferred_element_type=F32, precision=jax.lax.Precision.HIGHEST) + b_ref[...]
    out_refs[0][...] = y
    if stats:
        st_ref = out_refs[1]
        s = jnp.sum(y, axis=0, keepdims=True)
        qq = jnp.sum(y * y, axis=0, keepdims=True)

        @pl.when(pl.program_id(0) == 0)
        def _():
            st_ref[...] = jnp.zeros_like(st_ref)

        st_ref[...] = st_ref[...] + jnp.concatenate([s, qq], axis=0)


def _mlp_layer(x, st_in, g, beta, rows, W, b, tb, stats, act=True):
    R, cin = x.shape
    cout = W.shape[1]
    grid = (R // tb,)
    out_shape = [jax.ShapeDtypeStruct((R, cout), F32)]
    out_specs = [pl.BlockSpec((tb, cout), lambda i: (i, 0))]
    if stats:
        out_shape.append(jax.ShapeDtypeStruct((2, cout), F32))
        out_specs.append(pl.BlockSpec((2, cout), lambda i: (0, 0)))
    if st_in is None:
        st_in = jnp.zeros((2, cin), F32)
        g = jnp.ones((cin,), F32)
        beta = jnp.zeros((cin,), F32)
    res = pl.pallas_call(
        functools.partial(_mlp_body, act=act, stats=stats, rows=rows),
        grid=grid,
        in_specs=[
            pl.BlockSpec((tb, cin), lambda i: (i, 0)),
            pl.BlockSpec((2, cin), lambda i: (0, 0)),
            pl.BlockSpec((1, cin), lambda i: (0, 0)),
            pl.BlockSpec((1, cin), lambda i: (0, 0)),
            pl.BlockSpec((cin, cout), lambda i: (0, 0)),
            pl.BlockSpec((1, cout), lambda i: (0, 0)),
        ],
        out_specs=out_specs,
        out_shape=out_shape,
        compiler_params=pltpu.CompilerParams(
            dimension_semantics=("arbitrary",)),
    )(x, st_in, g[None, :], beta[None, :], W, b[None, :])
    if stats:
        return res[0], res[1]
    return res[0]


def _mlp2l1_body(a_ref, b_ref, c_ref, w_ref, bias_ref, y_ref, st_ref):
    w = w_ref[...]
    oc = a_ref.shape[1]
    hp = jax.lax.Precision.HIGHEST
    y = (jnp.dot(a_ref[...], w[0:oc], preferred_element_type=F32, precision=hp)
         + jnp.dot(b_ref[...], w[oc:2 * oc], preferred_element_type=F32, precision=hp)
         + jnp.dot(c_ref[...], w[2 * oc:3 * oc], preferred_element_type=F32, precision=hp)
         + bias_ref[...])
    y_ref[...] = y
    s = jnp.sum(y, axis=0, keepdims=True)
    qq = jnp.sum(y * y, axis=0, keepdims=True)

    @pl.when(pl.program_id(0) == 0)
    def _():
        st_ref[...] = jnp.zeros_like(st_ref)

    st_ref[...] = st_ref[...] + jnp.concatenate([s, qq], axis=0)


def _mlp2l1(a, b, c, W, bias, tb):
    R, oc = a.shape
    cout = W.shape[1]
    rspec = pl.BlockSpec((tb, oc), lambda i: (i, 0))
    return pl.pallas_call(
        _mlp2l1_body,
        grid=(R // tb,),
        in_specs=[rspec, rspec, rspec,
                  pl.BlockSpec((3 * oc, cout), lambda i: (0, 0)),
                  pl.BlockSpec((1, cout), lambda i: (0, 0))],
        out_specs=[pl.BlockSpec((tb, cout), lambda i: (i, 0)),
                   pl.BlockSpec((2, cout), lambda i: (0, 0))],
        out_shape=[jax.ShapeDtypeStruct((R, cout), F32),
                   jax.ShapeDtypeStruct((2, cout), F32)],
        compiler_params=pltpu.CompilerParams(
            dimension_semantics=("arbitrary",)),
    )(a, b, c, W, bias[None, :])


def _pf_body(x_ref, st_ref_in, g_ref, bt_ref, w4_ref, b4_ref, warr_ref,
             out_ref, *, rows):
    w = warr_ref[0]
    sc, sh = _bn_affine_in(st_ref_in[...], rows, g_ref[...], bt_ref[...])
    acc = jnp.zeros((x_ref.shape[2], x_ref.shape[3]), F32)
    for k in range(_KNN - 1):
        h = jnp.maximum(x_ref[0, k] * sc + sh, 0.0)
        acc = acc + h * w[:, k + 1:k + 2]
    out_ref[0] = (jnp.dot(acc, w4_ref[...], preferred_element_type=F32, precision=jax.lax.Precision.HIGHEST)
                  + w[:, 0:1] * b4_ref[...])


def kernel(pcl, pcl_noise, feature, params):
    B, N, _ = pcl_noise.shape
    M = pcl.shape[1]
    OC = feature.shape[-1]
    p7, p2 = params["mlp7"], params["mlp2"]
    TN = 512

    qpad = jnp.pad(pcl_noise, ((0, 0), (0, 0), (0, 5)))
    p8 = jnp.pad(pcl, ((0, 0), (0, 0), (0, 5)))
    pT8 = jnp.transpose(p8, (0, 2, 1))
    BF16 = jnp.bfloat16
    p_hi = p8.astype(BF16)
    r1 = p8 - p_hi.astype(F32)
    p_mid = r1.astype(BF16)
    p_lo = (r1 - p_mid.astype(F32)).astype(BF16)
    ps = jnp.concatenate([p_hi, p_mid, p_lo], axis=-1)
    f_hi = feature.astype(BF16)
    f_lo = (feature - f_hi.astype(F32)).astype(BF16)
    F2 = jnp.concatenate([f_hi, f_lo], axis=-1)
    W1 = p7["W"][0]
    c1 = W1.shape[1]
    w1a = jnp.pad(W1[0:3], ((0, 5), (0, 0)))
    w1b = jnp.pad(W1[3:6], ((0, 5), (0, 0)))

    x1, warr, cof, clf, st1 = pl.pallas_call(
        _knn_body,
        grid=(B, N // TN),
        in_specs=[
            pl.BlockSpec((1, TN, 8), lambda b, t: (b, t, 0)),
            pl.BlockSpec((1, 8, M), lambda b, t: (b, 0, 0)),
            pl.BlockSpec((1, M, 24), lambda b, t: (b, 0, 0)),
            pl.BlockSpec((1, M, 2 * OC), lambda b, t: (b, 0, 0)),
            pl.BlockSpec((8, c1), lambda b, t: (0, 0)),
            pl.BlockSpec((8, c1), lambda b, t: (0, 0)),
            pl.BlockSpec((1, c1), lambda b, t: (0, 0)),
        ],
        out_specs=[
            pl.BlockSpec((1, _KNN - 1, TN, c1), lambda b, t: (b, 0, t, 0)),
            pl.BlockSpec((1, TN, _KNN), lambda b, t: (b, t, 0)),
            pl.BlockSpec((1, TN, OC), lambda b, t: (b, t, 0)),
            pl.BlockSpec((1, TN, OC), lambda b, t: (b, t, 0)),
            pl.BlockSpec((2, c1), lambda b, t: (0, 0)),
        ],
        out_shape=[
            jax.ShapeDtypeStruct((B, _KNN - 1, N, c1), F32),
            jax.ShapeDtypeStruct((B, N, _KNN), F32),
            jax.ShapeDtypeStruct((B, N, OC), F32),
            jax.ShapeDtypeStruct((B, N, OC), F32),
            jax.ShapeDtypeStruct((2, c1), F32),
        ],
        compiler_params=pltpu.CompilerParams(
            dimension_semantics=("arbitrary", "arbitrary")),
    )(qpad, pT8, ps, F2, w1a, w1b, p7["b"][0][None, :])

    R1 = B * (_KNN - 1) * N
    x2, st2 = _mlp_layer(x1.reshape(R1, c1), st1, p7["g"][0], p7["beta"][0],
                         float(R1), p7["W"][1], p7["b"][1], tb=2048, stats=True)
    x3, st3 = _mlp_layer(x2, st2, p7["g"][1], p7["beta"][1],
                         float(R1), p7["W"][2], p7["b"][2], tb=2048, stats=True)
    c3 = p7["W"][2].shape[1]

    pf = pl.pallas_call(
        functools.partial(_pf_body, rows=float(R1)),
        grid=(B, N // TN),
        in_specs=[
            pl.BlockSpec((1, _KNN - 1, TN, c3), lambda b, t: (b, 0, t, 0)),
            pl.BlockSpec((2, c3), lambda b, t: (0, 0)),
            pl.BlockSpec((1, c3), lambda b, t: (0, 0)),
            pl.BlockSpec((1, c3), lambda b, t: (0, 0)),
            pl.BlockSpec((c3, OC), lambda b, t: (0, 0)),
            pl.BlockSpec((1, OC), lambda b, t: (0, 0)),
            pl.BlockSpec((1, TN, _KNN), lambda b, t: (b, t, 0)),
        ],
        out_specs=pl.BlockSpec((1, TN, OC), lambda b, t: (b, t, 0)),
        out_shape=jax.ShapeDtypeStruct((B, N, OC), F32),
        compiler_params=pltpu.CompilerParams(
            dimension_semantics=("arbitrary", "arbitrary")),
    )(x3.reshape(B, _KNN - 1, N, c3), st3, p7["g"][2][None, :],
      p7["beta"][2][None, :], p7["W"][3], p7["b"][3][None, :], warr)

    R2 = B * N
    x4, st4 = _mlp2l1(clf.reshape(R2, OC), cof.reshape(R2, OC),
                      pf.reshape(R2, OC), p2["W"][0], p2["b"][0], tb=1024)
    x5, st5 = _mlp_layer(x4, st4, p2["g"][0], p2["beta"][0],
                         float(R2), p2["W"][1], p2["b"][1], tb=1024, stats=True)
    x6, st6 = _mlp_layer(x5, st5, p2["g"][1], p2["beta"][1],
                         float(R2), p2["W"][2], p2["b"][2], tb=1024, stats=True)
    out = _mlp_layer(x6, st6, p2["g"][2], p2["beta"][2],
                     float(R2), p2["W"][3], p2["b"][3], tb=1024, stats=False)
    return out.reshape(B, N, OC)


# speedup vs baseline: 1.0248x; 1.0248x over previous
import functools

import jax
import jax.numpy as jnp
from jax.experimental import pallas as pl
from jax.experimental.pallas import tpu as pltpu

F32 = jnp.float32
_BN_EPS = 1e-5
_KNN = 8


def _knn_body(q_ref, pT_ref, ps_ref, f2_ref, w1a_ref, w1b_ref, b1_ref,
              x1_ref, warr_ref, cof_ref, clf_ref, st_ref):
    TN = q_ref.shape[1]
    M = pT_ref.shape[2]
    OC = f2_ref.shape[2] // 2
    q = q_ref[0]
    pT = pT_ref[0]
    ps = ps_ref[0]
    F2 = f2_ref[0]
    pn = jnp.sum(pT * pT, axis=0, keepdims=True)
    iota = jax.lax.broadcasted_iota(jnp.int32, (TN, M), 1)

    def amin(d2):
        m = jnp.min(d2, axis=1, keepdims=True)
        return jnp.min(jnp.where(d2 == m, iota, M), axis=1, keepdims=True)

    def dist2(v):
        acc = v[:, 0:1] * pT[0:1, :]
        acc = acc + v[:, 1:2] * pT[1:2, :]
        acc = acc + v[:, 2:3] * pT[2:3, :]
        return pn - 2.0 * acc

    def gather24(oh_bf):
        g = jnp.dot(oh_bf, ps, preferred_element_type=F32)
        return (g[:, 16:24] + g[:, 8:16]) + g[:, 0:8]

    ci = amin(dist2(q))
    ohcb = (iota == ci).astype(F32).astype(jnp.bfloat16)
    cp = gather24(ohcb)
    clf2 = jnp.dot(ohcb, F2, preferred_element_type=F32)
    clf_ref[0] = clf2[:, :OC] + clf2[:, OC:]

    d2 = dist2(cp)
    emat = jnp.zeros((TN, M), F32)
    es = []
    deltas = []
    for k in range(_KNN):
        ki = amin(d2)
        oh = iota == ki
        d2 = jnp.where(oh, jnp.inf, d2)
        if k == 0:
            continue
        ohf = oh.astype(F32)
        delta = gather24(ohf.astype(jnp.bfloat16)) - q
        dst = jnp.sqrt(jnp.sum(delta * delta, axis=1, keepdims=True))
        e = jnp.exp(-10.0 * dst)
        emat = emat + e * ohf
        es.append(e)
        deltas.append(delta)

    esum = es[0]
    for e in es[1:]:
        esum = esum + e
    inv = 1.0 / (esum + 1e-7)
    wmat = emat * inv
    wh = wmat.astype(jnp.bfloat16)
    wl = (wmat - wh.astype(F32)).astype(jnp.bfloat16)
    y1 = jnp.dot(wh, F2, preferred_element_type=F32)
    y2 = jnp.dot(wl, F2[:, :OC], preferred_element_type=F32)
    cof_ref[0] = (y1[:, :OC] + y1[:, OC:]) + y2
    warr_ref[0] = jnp.concatenate([esum * inv] + [e * inv for e in es], axis=1)

    base = jnp.dot(cp, w1b_ref[...], preferred_element_type=F32, precision=jax.lax.Precision.HIGHEST) + b1_ref[...]
    ssum = jnp.zeros((1, x1_ref.shape[-1]), F32)
    qsum = jnp.zeros((1, x1_ref.shape[-1]), F32)
    for k in range(_KNN - 1):
        x1 = jnp.dot(deltas[k], w1a_ref[...], preferred_element_type=F32, precision=jax.lax.Precision.HIGHEST) + base
        x1_ref[0, k] = x1
        ssum = ssum + jnp.sum(x1, axis=0, keepdims=True)
        qsum = qsum + jnp.sum(x1 * x1, axis=0, keepdims=True)

    first = jnp.logical_and(pl.program_id(0) == 0, pl.program_id(1) == 0)

    @pl.when(first)
    def _():
        st_ref[...] = jnp.zeros_like(st_ref)

    st_ref[...] = st_ref[...] + jnp.concatenate([ssum, qsum], axis=0)


def _bn_affine_in(st, rows, g, beta):
    mu = st[0:1] / rows
    var = jnp.maximum(st[1:2] / rows - mu * mu, 0.0)
    sc = g * jax.lax.rsqrt(var + _BN_EPS)
    sh = beta - mu * sc
    return sc, sh


def _mlp_body(x_ref, st_ref_in, g_ref, bt_ref, w_ref, b_ref, *out_refs,
              act, stats, rows):
    if act:
        sc, sh = _bn_affine_in(st_ref_in[...], rows, g_ref[...], bt_ref[...])
        h = jnp.maximum(x_ref[...] * sc + sh, 0.0)
    else:
        h = x_ref[...]
    y = jnp.dot(h, w_ref[...], preferred_element_type=F32, precision=jax.lax.Precision.HIGHEST) + b_ref[...]
    out_refs[0][...] = y
    if stats:
        st_ref = out_refs[1]
        s = jnp.sum(y, axis=0, keepdims=True)
        qq = jnp.sum(y * y, axis=0, keepdims=True)

        @pl.when(pl.program_id(0) == 0)
        def _():
            st_ref[...] = jnp.zeros_like(st_ref)

        st_ref[...] = st_ref[...] + jnp.concatenate([s, qq], axis=0)


def _mlp_layer(x, st_in, g, beta, rows, W, b, tb, stats, act=True):
    R, cin = x.shape
    cout = W.shape[1]
    grid = (R // tb,)
    out_shape = [jax.ShapeDtypeStruct((R, cout), F32)]
    out_specs = [pl.BlockSpec((tb, cout), lambda i: (i, 0))]
    if stats:
        out_shape.append(jax.ShapeDtypeStruct((2, cout), F32))
        out_specs.append(pl.BlockSpec((2, cout), lambda i: (0, 0)))
    if st_in is None:
        st_in = jnp.zeros((2, cin), F32)
        g = jnp.ones((cin,), F32)
        beta = jnp.zeros((cin,), F32)
    res = pl.pallas_call(
        functools.partial(_mlp_body, act=act, stats=stats, rows=rows),
        grid=grid,
        in_specs=[
            pl.BlockSpec((tb, cin), lambda i: (i, 0)),
            pl.BlockSpec((2, cin), lambda i: (0, 0)),
            pl.BlockSpec((1, cin), lambda i: (0, 0)),
            pl.BlockSpec((1, cin), lambda i: (0, 0)),
            pl.BlockSpec((cin, cout), lambda i: (0, 0)),
            pl.BlockSpec((1, cout), lambda i: (0, 0)),
        ],
        out_specs=out_specs,
        out_shape=out_shape,
        compiler_params=pltpu.CompilerParams(
            dimension_semantics=("arbitrary",)),
    )(x, st_in, g[None, :], beta[None, :], W, b[None, :])
    if stats:
        return res[0], res[1]
    return res[0]


def _pf_body(x_ref, st_ref_in, g_ref, bt_ref, w4_ref, b4_ref, warr_ref,
             out_ref, *, rows):
    w = warr_ref[0]
    sc, sh = _bn_affine_in(st_ref_in[...], rows, g_ref[...], bt_ref[...])
    acc = jnp.zeros((x_ref.shape[2], x_ref.shape[3]), F32)
    for k in range(_KNN - 1):
        h = jnp.maximum(x_ref[0, k] * sc + sh, 0.0)
        acc = acc + h * w[:, k + 1:k + 2]
    out_ref[0] = (jnp.dot(acc, w4_ref[...], preferred_element_type=F32, precision=jax.lax.Precision.HIGHEST)
                  + w[:, 0:1] * b4_ref[...])


def kernel(pcl, pcl_noise, feature, params):
    B, N, _ = pcl_noise.shape
    M = pcl.shape[1]
    OC = feature.shape[-1]
    p7, p2 = params["mlp7"], params["mlp2"]
    TN = 512

    qpad = jnp.pad(pcl_noise, ((0, 0), (0, 0), (0, 5)))
    p8 = jnp.pad(pcl, ((0, 0), (0, 0), (0, 5)))
    pT8 = jnp.transpose(p8, (0, 2, 1))
    BF16 = jnp.bfloat16
    p_hi = p8.astype(BF16)
    r1 = p8 - p_hi.astype(F32)
    p_mid = r1.astype(BF16)
    p_lo = (r1 - p_mid.astype(F32)).astype(BF16)
    ps = jnp.concatenate([p_hi, p_mid, p_lo], axis=-1)
    f_hi = feature.astype(BF16)
    f_lo = (feature - f_hi.astype(F32)).astype(BF16)
    F2 = jnp.concatenate([f_hi, f_lo], axis=-1)
    W1 = p7["W"][0]
    c1 = W1.shape[1]
    w1a = jnp.pad(W1[0:3], ((0, 5), (0, 0)))
    w1b = jnp.pad(W1[3:6], ((0, 5), (0, 0)))

    x1, warr, cof, clf, st1 = pl.pallas_call(
        _knn_body,
        grid=(B, N // TN),
        in_specs=[
            pl.BlockSpec((1, TN, 8), lambda b, t: (b, t, 0)),
            pl.BlockSpec((1, 8, M), lambda b, t: (b, 0, 0)),
            pl.BlockSpec((1, M, 24), lambda b, t: (b, 0, 0)),
            pl.BlockSpec((1, M, 2 * OC), lambda b, t: (b, 0, 0)),
            pl.BlockSpec((8, c1), lambda b, t: (0, 0)),
            pl.BlockSpec((8, c1), lambda b, t: (0, 0)),
            pl.BlockSpec((1, c1), lambda b, t: (0, 0)),
        ],
        out_specs=[
            pl.BlockSpec((1, _KNN - 1, TN, c1), lambda b, t: (b, 0, t, 0)),
            pl.BlockSpec((1, TN, _KNN), lambda b, t: (b, t, 0)),
            pl.BlockSpec((1, TN, OC), lambda b, t: (b, t, 0)),
            pl.BlockSpec((1, TN, OC), lambda b, t: (b, t, 0)),
            pl.BlockSpec((2, c1), lambda b, t: (0, 0)),
        ],
        out_shape=[
            jax.ShapeDtypeStruct((B, _KNN - 1, N, c1), F32),
            jax.ShapeDtypeStruct((B, N, _KNN), F32),
            jax.ShapeDtypeStruct((B, N, OC), F32),
            jax.ShapeDtypeStruct((B, N, OC), F32),
            jax.ShapeDtypeStruct((2, c1), F32),
        ],
        compiler_params=pltpu.CompilerParams(
            dimension_semantics=("arbitrary", "arbitrary")),
    )(qpad, pT8, ps, F2, w1a, w1b, p7["b"][0][None, :])

    R1 = B * (_KNN - 1) * N
    x2, st2 = _mlp_layer(x1.reshape(R1, c1), st1, p7["g"][0], p7["beta"][0],
                         float(R1), p7["W"][1], p7["b"][1], tb=4096, stats=True)
    x3, st3 = _mlp_layer(x2, st2, p7["g"][1], p7["beta"][1],
                         float(R1), p7["W"][2], p7["b"][2], tb=4096, stats=True)
    c3 = p7["W"][2].shape[1]

    pf = pl.pallas_call(
        functools.partial(_pf_body, rows=float(R1)),
        grid=(B, N // TN),
        in_specs=[
            pl.BlockSpec((1, _KNN - 1, TN, c3), lambda b, t: (b, 0, t, 0)),
            pl.BlockSpec((2, c3), lambda b, t: (0, 0)),
            pl.BlockSpec((1, c3), lambda b, t: (0, 0)),
            pl.BlockSpec((1, c3), lambda b, t: (0, 0)),
            pl.BlockSpec((c3, OC), lambda b, t: (0, 0)),
            pl.BlockSpec((1, OC), lambda b, t: (0, 0)),
            pl.BlockSpec((1, TN, _KNN), lambda b, t: (b, t, 0)),
        ],
        out_specs=pl.BlockSpec((1, TN, OC), lambda b, t: (b, t, 0)),
        out_shape=jax.ShapeDtypeStruct((B, N, OC), F32),
        compiler_params=pltpu.CompilerParams(
            dimension_semantics=("arbitrary", "arbitrary")),
    )(x3.reshape(B, _KNN - 1, N, c3), st3, p7["g"][2][None, :],
      p7["beta"][2][None, :], p7["W"][3], p7["b"][3][None, :], warr)

    R2 = B * N
    df = jnp.concatenate([clf, cof, pf], axis=-1).reshape(R2, 3 * OC)
    x4, st4 = _mlp_layer(df, None, None, None,
                         float(R2), p2["W"][0], p2["b"][0],
                         tb=1024, stats=True, act=False)
    x5, st5 = _mlp_layer(x4, st4, p2["g"][0], p2["beta"][0],
                         float(R2), p2["W"][1], p2["b"][1], tb=1024, stats=True)
    x6, st6 = _mlp_layer(x5, st5, p2["g"][1], p2["beta"][1],
                         float(R2), p2["W"][2], p2["b"][2], tb=1024, stats=True)
    out = _mlp_layer(x6, st6, p2["g"][2], p2["beta"][2],
                     float(R2), p2["W"][3], p2["b"][3], tb=1024, stats=False)
    return out.reshape(B, N, OC)
